# Initial kernel scaffold; baseline (speedup 1.0000x reference)
import jax
import jax.numpy as jnp
from jax.experimental import pallas as pl


def kernel(x_author, x_paper, x_term, edge_index, edge_type, Wa, ba, Wp, bp, Wt, bt, basis1, comp1, root1, bias1, basis2, comp2, root2, bias2):
    raise NotImplementedError("write your pallas kernel here")


# speedup vs baseline: 10.6499x; 10.6499x over previous
import functools

import jax
import jax.numpy as jnp
from jax import lax
from jax.experimental import pallas as pl
from jax.experimental.pallas import tpu as pltpu
from jax.experimental.pallas import tpu_sc as plsc

NR = 5
D = 128
NC = 2
NS = 16
LANES = 16
CHUNK = 128


def _mm_bias(x, w, b2d):
    def body(x_ref, w_ref, b_ref, o_ref):
        o_ref[...] = jnp.dot(x_ref[...], w_ref[...],
                             preferred_element_type=jnp.float32) + b_ref[...]
    return pl.pallas_call(
        body,
        out_shape=jax.ShapeDtypeStruct((x.shape[0], w.shape[1]), jnp.float32),
    )(x, w, b2d)


def _comp_basis(comp, basis2d):
    def body(c_ref, b_ref, o_ref):
        o_ref[...] = jnp.dot(c_ref[...], b_ref[...],
                             preferred_element_type=jnp.float32)
    return pl.pallas_call(
        body,
        out_shape=jax.ShapeDtypeStruct((comp.shape[0], basis2d.shape[1]),
                                       jnp.float32),
    )(comp, basis2d)


def _h_all(x, wall, b2d):
    n = x.shape[0]
    blk = 1000

    def body(x_ref, w_ref, b_ref, o_ref):
        r = pl.program_id(0)
        h = jnp.dot(x_ref[...], w_ref[0], preferred_element_type=jnp.float32)
        o_ref[0] = h + jnp.where(r == NR, 1.0, 0.0) * b_ref[...]

    return pl.pallas_call(
        body,
        grid=(NR + 1, n // blk),
        in_specs=[
            pl.BlockSpec((blk, D), lambda r, i: (i, 0)),
            pl.BlockSpec((1, D, D), lambda r, i: (r, 0, 0)),
            pl.BlockSpec((1, D), lambda r, i: (0, 0)),
        ],
        out_specs=pl.BlockSpec((1, blk, D), lambda r, i: (r, i, 0)),
        out_shape=jax.ShapeDtypeStruct((NR + 1, n, D), jnp.float32),
    )(x, wall, b2d)


def _add3(a, b, c):
    n = a.shape[0]
    blk = 1000 if n % 1000 == 0 else n

    def body(a_ref, b_ref, c_ref, o_ref):
        o_ref[...] = a_ref[...] + b_ref[...] + c_ref[...]

    return pl.pallas_call(
        body,
        grid=(n // blk,),
        in_specs=[pl.BlockSpec((blk, D), lambda i: (i, 0))] * 3,
        out_specs=pl.BlockSpec((blk, D), lambda i: (i, 0)),
        out_shape=jax.ShapeDtypeStruct((n, D), jnp.float32),
    )(a, b, c)


@functools.lru_cache(maxsize=None)
def _edge_prep_kernel(E, N):
    nchunks = E // CHUNK
    cnt_size = NR * N
    stripe = ((cnt_size + NS * 128 - 1) // (NS * 128)) * 128
    cnt_pad = stripe * NS
    iters_core = (nchunks + NS - 1) // NS
    iters_all = (nchunks + NS * NC - 1) // (NS * NC)
    mesh = plsc.VectorSubcoreMesh(core_axis_name="c", subcore_axis_name="s")

    def body(src_hbm, dst_hbm, et_hbm, w_hbm, g_hbm,
             cnt_sp, tbl_v, a_v, b_v, idx_v, f_v, sem):
        s = lax.axis_index("s")
        c = lax.axis_index("c")
        wid = s * NC + c

        def z(i, _):
            tbl_v[pl.ds(i * LANES, LANES)] = jnp.zeros((LANES,), jnp.float32)
            return 0
        lax.fori_loop(0, stripe // LANES, z, 0)
        pltpu.sync_copy(tbl_v.at[pl.ds(0, stripe)],
                        cnt_sp.at[pl.ds(s * stripe, stripe)])
        plsc.subcore_barrier()

        for j in range(CHUNK // LANES):
            f_v[pl.ds(j * LANES, LANES)] = jnp.ones((LANES,), jnp.float32)

        def p1(i, _):
            ch = s + i * NS
            @pl.when(ch < nchunks)
            def _():
                off = ch * CHUNK
                pltpu.sync_copy(et_hbm.at[pl.ds(off, CHUNK)], a_v)
                pltpu.sync_copy(dst_hbm.at[pl.ds(off, CHUNK)], b_v)
                for j in range(CHUNK // LANES):
                    sl = pl.ds(j * LANES, LANES)
                    idx_v[sl] = a_v[sl] * N + b_v[sl]
                pltpu.sync_copy(f_v, cnt_sp.at[idx_v], add=True)
            return 0
        lax.fori_loop(0, iters_core, p1, 0)
        plsc.subcore_barrier()

        pltpu.sync_copy(cnt_sp.at[pl.ds(s * stripe, stripe)],
                        tbl_v.at[pl.ds(0, stripe)])
        def inv(i, _):
            sl = pl.ds(i * LANES, LANES)
            tbl_v[sl] = 1.0 / jnp.maximum(tbl_v[sl], 1.0)
            return 0
        lax.fori_loop(0, stripe // LANES, inv, 0)
        pltpu.sync_copy(tbl_v.at[pl.ds(0, stripe)],
                        cnt_sp.at[pl.ds(s * stripe, stripe)])
        plsc.subcore_barrier()

        def p2(i, _):
            ch = wid + i * NS * NC
            @pl.when(ch < nchunks)
            def _():
                off = ch * CHUNK
                pltpu.sync_copy(et_hbm.at[pl.ds(off, CHUNK)], a_v)
                pltpu.sync_copy(src_hbm.at[pl.ds(off, CHUNK)], b_v)
                for j in range(CHUNK // LANES):
                    sl = pl.ds(j * LANES, LANES)
                    idx_v[sl] = a_v[sl] * N + b_v[sl]
                pltpu.sync_copy(idx_v, g_hbm.at[pl.ds(off, CHUNK)])
                pltpu.sync_copy(dst_hbm.at[pl.ds(off, CHUNK)], b_v)
                for j in range(CHUNK // LANES):
                    sl = pl.ds(j * LANES, LANES)
                    idx_v[sl] = a_v[sl] * N + b_v[sl]
                pltpu.async_copy(cnt_sp.at[idx_v], f_v, sem).wait()
                pltpu.sync_copy(f_v, w_hbm.at[pl.ds(off, CHUNK)])
            return 0
        lax.fori_loop(0, iters_all, p2, 0)

    return pl.kernel(
        body,
        out_type=[jax.ShapeDtypeStruct((E,), jnp.float32),
                  jax.ShapeDtypeStruct((E,), jnp.int32)],
        mesh=mesh,
        scratch_types=[
            pltpu.VMEM_SHARED((cnt_pad,), jnp.float32),
            pltpu.VMEM((stripe,), jnp.float32),
            pltpu.VMEM((CHUNK,), jnp.int32),
            pltpu.VMEM((CHUNK,), jnp.int32),
            pltpu.VMEM((CHUNK,), jnp.int32),
            pltpu.VMEM((CHUNK,), jnp.float32),
            pltpu.SemaphoreType.DMA,
        ],
    )


@functools.lru_cache(maxsize=None)
def _msg_kernel(E, N):
    nchunks = E // CHUNK
    iters = (nchunks + NS * NC - 1) // (NS * NC)
    sub = 80
    nsub = N // sub
    subiters = (nsub + NS - 1) // NS
    mesh = plsc.VectorSubcoreMesh(core_axis_name="c", subcore_axis_name="s")

    def body(h_hbm, g_hbm, dst_hbm, w_hbm, out_hbm,
             acc_sp, rows_v, zbuf_v, g_v, dst_v, w_v, sem):
        s = lax.axis_index("s")
        c = lax.axis_index("c")
        wid = s * NC + c

        def z(i, _):
            for j in range(D // LANES):
                zbuf_v[i, pl.ds(j * LANES, LANES)] = jnp.zeros((LANES,),
                                                               jnp.float32)
            return 0
        lax.fori_loop(0, sub, z, 0)

        def z2(i, _):
            k = s + i * NS
            @pl.when(k < nsub)
            def _():
                pltpu.sync_copy(zbuf_v, acc_sp.at[pl.ds(k * sub, sub)])
            return 0
        lax.fori_loop(0, subiters, z2, 0)
        plsc.subcore_barrier()

        def step(i, _):
            ch = wid + i * NS * NC
            @pl.when(ch < nchunks)
            def _():
                off = ch * CHUNK
                pltpu.sync_copy(g_hbm.at[pl.ds(off, CHUNK)], g_v)
                gather = pltpu.async_copy(h_hbm.at[g_v], rows_v, sem)
                pltpu.sync_copy(w_hbm.at[pl.ds(off, CHUNK)], w_v.at[pl.ds(0, CHUNK)])
                pltpu.sync_copy(dst_hbm.at[pl.ds(off, CHUNK)], dst_v)
                gather.wait()

                def scale(e, _):
                    wv = w_v[pl.ds(e, LANES)][0]
                    for j in range(D // LANES):
                        sl = pl.ds(j * LANES, LANES)
                        rows_v[e, sl] = rows_v[e, sl] * wv
                    return 0
                lax.fori_loop(0, CHUNK, scale, 0)
                pltpu.sync_copy(rows_v, acc_sp.at[dst_v], add=True)
            return 0
        lax.fori_loop(0, iters, step, 0)
        plsc.subcore_barrier()

        def drain(i, _):
            k = s + i * NS
            @pl.when(k < nsub)
            def _():
                pltpu.sync_copy(acc_sp.at[pl.ds(k * sub, sub)], zbuf_v)
                pltpu.sync_copy(zbuf_v, out_hbm.at[c, pl.ds(k * sub, sub)])
            return 0
        lax.fori_loop(0, subiters, drain, 0)

    return pl.kernel(
        body,
        out_type=jax.ShapeDtypeStruct((NC, N, D), jnp.float32),
        mesh=mesh,
        scratch_types=[
            pltpu.VMEM_SHARED((N, D), jnp.float32),
            pltpu.VMEM((CHUNK, D), jnp.float32),
            pltpu.VMEM((sub, D), jnp.float32),
            pltpu.VMEM((CHUNK,), jnp.int32),
            pltpu.VMEM((CHUNK,), jnp.int32),
            pltpu.VMEM((CHUNK + LANES,), jnp.float32),
            pltpu.SemaphoreType.DMA,
        ],
    )


@jax.jit
def _impl(x_author, x_paper, x_term, edge_index, edge_type,
          Wa, ba, Wp, bp, Wt, bt,
          basis1, comp1, root1, bias1,
          basis2, comp2, root2, bias2):
    num_out = x_author.shape[0]
    N = x_author.shape[0] + x_paper.shape[0] + x_term.shape[0]
    E = edge_index.shape[1]
    d_in = root1.shape[0]

    w1 = _comp_basis(comp1, basis1.reshape(basis1.shape[0], -1))
    w1all = jnp.concatenate([w1.reshape(NR, d_in, D), root1[None]], axis=0)
    w2 = _comp_basis(comp2, basis2.reshape(basis2.shape[0], -1))
    w2all = jnp.concatenate([w2.reshape(NR, d_in, D), root2[None]], axis=0)

    xa = _mm_bias(x_author, Wa, ba[None, :])
    xp = _mm_bias(x_paper, Wp, bp[None, :])
    xt = _mm_bias(x_term, Wt, bt[None, :])
    x = jnp.concatenate([xa, xp, xt], axis=0)

    src = edge_index[0]
    dst = edge_index[1]
    w_e, g_e = _edge_prep_kernel(E, N)(src, dst, edge_type)

    h1 = _h_all(x, w1all, bias1[None, :])
    acc1 = _msg_kernel(E, N)(h1.reshape((NR + 1) * N, D), g_e, dst, w_e)
    x2 = _add3(h1[NR], acc1[0], acc1[1])

    h2 = _h_all(x2, w2all, bias2[None, :])
    acc2 = _msg_kernel(E, N)(h2.reshape((NR + 1) * N, D), g_e, dst, w_e)
    out = _add3(h2[NR, :num_out], acc2[0, :num_out], acc2[1, :num_out])
    return out


def kernel(x_author, x_paper, x_term, edge_index, edge_type,
           Wa, ba, Wp, bp, Wt, bt,
           basis1, comp1, root1, bias1,
           basis2, comp2, root2, bias2):
    return _impl(x_author, x_paper, x_term, edge_index, edge_type,
                 Wa, ba, Wp, bp, Wt, bt,
                 basis1, comp1, root1, bias1,
                 basis2, comp2, root2, bias2)


# speedup vs baseline: 14.8398x; 1.3934x over previous
import functools

import jax
import jax.numpy as jnp
from jax import lax
from jax.experimental import pallas as pl
from jax.experimental.pallas import tpu as pltpu
from jax.experimental.pallas import tpu_sc as plsc

NR = 5
D = 128
NC = 2
NS = 16
LANES = 16
CHUNK = 128


def _mm_bias(x, w, b2d):
    def body(x_ref, w_ref, b_ref, o_ref):
        o_ref[...] = jnp.dot(x_ref[...], w_ref[...],
                             preferred_element_type=jnp.float32) + b_ref[...]
    return pl.pallas_call(
        body,
        out_shape=jax.ShapeDtypeStruct((x.shape[0], w.shape[1]), jnp.float32),
    )(x, w, b2d)


def _comp_basis(comp, basis2d):
    def body(c_ref, b_ref, o_ref):
        o_ref[...] = jnp.dot(c_ref[...], b_ref[...],
                             preferred_element_type=jnp.float32)
    return pl.pallas_call(
        body,
        out_shape=jax.ShapeDtypeStruct((comp.shape[0], basis2d.shape[1]),
                                       jnp.float32),
    )(comp, basis2d)


def _h_all(x, wall, b2d):
    n = x.shape[0]
    blk = 1000

    def body(x_ref, w_ref, b_ref, o_ref):
        r = pl.program_id(0)
        h = jnp.dot(x_ref[...], w_ref[0], preferred_element_type=jnp.float32)
        o_ref[0] = h + jnp.where(r == NR, 1.0, 0.0) * b_ref[...]

    return pl.pallas_call(
        body,
        grid=(NR + 1, n // blk),
        in_specs=[
            pl.BlockSpec((blk, D), lambda r, i: (i, 0)),
            pl.BlockSpec((1, D, D), lambda r, i: (r, 0, 0)),
            pl.BlockSpec((1, D), lambda r, i: (0, 0)),
        ],
        out_specs=pl.BlockSpec((1, blk, D), lambda r, i: (r, i, 0)),
        out_shape=jax.ShapeDtypeStruct((NR + 1, n, D), jnp.float32),
    )(x, wall, b2d)


def _add3(a, b, c):
    n = a.shape[0]
    blk = 1000 if n % 1000 == 0 else n

    def body(a_ref, b_ref, c_ref, o_ref):
        o_ref[...] = a_ref[...] + b_ref[...] + c_ref[...]

    return pl.pallas_call(
        body,
        grid=(n // blk,),
        in_specs=[pl.BlockSpec((blk, D), lambda i: (i, 0))] * 3,
        out_specs=pl.BlockSpec((blk, D), lambda i: (i, 0)),
        out_shape=jax.ShapeDtypeStruct((n, D), jnp.float32),
    )(a, b, c)


@functools.lru_cache(maxsize=None)
def _edge_prep_kernel(E, N):
    nchunks = E // CHUNK
    cnt_size = NR * N
    stripe = ((cnt_size + NS * 128 - 1) // (NS * 128)) * 128
    cnt_pad = stripe * NS
    iters_core = (nchunks + NS - 1) // NS
    iters_all = (nchunks + NS * NC - 1) // (NS * NC)
    mesh = plsc.VectorSubcoreMesh(core_axis_name="c", subcore_axis_name="s")

    def body(src_hbm, dst_hbm, et_hbm, w_hbm, g_hbm,
             cnt_sp, tbl_v, a_v, b_v, idx_v, f_v, sem):
        s = lax.axis_index("s")
        c = lax.axis_index("c")
        wid = s * NC + c

        def z(i, _):
            tbl_v[pl.ds(i * LANES, LANES)] = jnp.zeros((LANES,), jnp.float32)
            return 0
        lax.fori_loop(0, stripe // LANES, z, 0)
        pltpu.sync_copy(tbl_v.at[pl.ds(0, stripe)],
                        cnt_sp.at[pl.ds(s * stripe, stripe)])
        plsc.subcore_barrier()

        for j in range(CHUNK // LANES):
            f_v[pl.ds(j * LANES, LANES)] = jnp.ones((LANES,), jnp.float32)

        def p1(i, _):
            ch = s + i * NS
            @pl.when(ch < nchunks)
            def _():
                off = ch * CHUNK
                pltpu.sync_copy(et_hbm.at[pl.ds(off, CHUNK)], a_v)
                pltpu.sync_copy(dst_hbm.at[pl.ds(off, CHUNK)], b_v)
                for j in range(CHUNK // LANES):
                    sl = pl.ds(j * LANES, LANES)
                    idx_v[sl] = a_v[sl] * N + b_v[sl]
                pltpu.sync_copy(f_v, cnt_sp.at[idx_v], add=True)
            return 0
        lax.fori_loop(0, iters_core, p1, 0)
        plsc.subcore_barrier()

        pltpu.sync_copy(cnt_sp.at[pl.ds(s * stripe, stripe)],
                        tbl_v.at[pl.ds(0, stripe)])
        def inv(i, _):
            sl = pl.ds(i * LANES, LANES)
            tbl_v[sl] = 1.0 / jnp.maximum(tbl_v[sl], 1.0)
            return 0
        lax.fori_loop(0, stripe // LANES, inv, 0)
        pltpu.sync_copy(tbl_v.at[pl.ds(0, stripe)],
                        cnt_sp.at[pl.ds(s * stripe, stripe)])
        plsc.subcore_barrier()

        def p2(i, _):
            ch = wid + i * NS * NC
            @pl.when(ch < nchunks)
            def _():
                off = ch * CHUNK
                pltpu.sync_copy(et_hbm.at[pl.ds(off, CHUNK)], a_v)
                pltpu.sync_copy(src_hbm.at[pl.ds(off, CHUNK)], b_v)
                for j in range(CHUNK // LANES):
                    sl = pl.ds(j * LANES, LANES)
                    idx_v[sl] = a_v[sl] * N + b_v[sl]
                pltpu.sync_copy(idx_v, g_hbm.at[pl.ds(off, CHUNK)])
                pltpu.sync_copy(dst_hbm.at[pl.ds(off, CHUNK)], b_v)
                for j in range(CHUNK // LANES):
                    sl = pl.ds(j * LANES, LANES)
                    idx_v[sl] = a_v[sl] * N + b_v[sl]
                pltpu.async_copy(cnt_sp.at[idx_v], f_v, sem).wait()
                pltpu.sync_copy(f_v, w_hbm.at[pl.ds(off, CHUNK)])
            return 0
        lax.fori_loop(0, iters_all, p2, 0)

    return pl.kernel(
        body,
        out_type=[jax.ShapeDtypeStruct((E,), jnp.float32),
                  jax.ShapeDtypeStruct((E,), jnp.int32)],
        mesh=mesh,
        scratch_types=[
            pltpu.VMEM_SHARED((cnt_pad,), jnp.float32),
            pltpu.VMEM((stripe,), jnp.float32),
            pltpu.VMEM((CHUNK,), jnp.int32),
            pltpu.VMEM((CHUNK,), jnp.int32),
            pltpu.VMEM((CHUNK,), jnp.int32),
            pltpu.VMEM((CHUNK,), jnp.float32),
            pltpu.SemaphoreType.DMA,
        ],
    )


@functools.lru_cache(maxsize=None)
def _msg_kernel(E, N):
    BLK = 128
    SUBC = BLK // CHUNK
    nblocks = E // BLK
    NW = NS * NC
    iters = (nblocks + NW - 1) // NW
    half = (iters + 1) // 2
    sub = 40
    nsub = N // sub
    subiters = (nsub + NS - 1) // NS
    mesh = plsc.VectorSubcoreMesh(core_axis_name="c", subcore_axis_name="s")

    def body(h_hbm, g_hbm, dst_hbm, w_hbm, out_hbm,
             acc_sp, rows_v, zbuf_v, gbuf, dbuf, wbuf,
             sem_l0, sem_l1, sem_g0, sem_g1):
        s = lax.axis_index("s")
        c = lax.axis_index("c")
        wid = s * NC + c
        sem_l = (sem_l0, sem_l1)
        sem_g = (sem_g0, sem_g1)

        def z(i, _):
            for j in range(D // LANES):
                zbuf_v[i, pl.ds(j * LANES, LANES)] = jnp.zeros((LANES,),
                                                               jnp.float32)
            return 0
        lax.fori_loop(0, sub, z, 0)

        def z2(i, _):
            k = s + i * NS
            @pl.when(k < nsub)
            def _():
                pltpu.sync_copy(zbuf_v, acc_sp.at[pl.ds(k * sub, sub)])
            return 0
        lax.fori_loop(0, subiters, z2, 0)
        plsc.subcore_barrier()

        def prefetch(k, b):
            ch = wid + k * NW
            @pl.when(ch < nblocks)
            def _():
                row0 = ch * SUBC
                d1 = pltpu.async_copy(g_hbm.at[pl.ds(row0, SUBC)],
                                      gbuf.at[b], sem_l[b])
                d2 = pltpu.async_copy(dst_hbm.at[pl.ds(row0, SUBC)],
                                      dbuf.at[b], sem_l[b])
                d3 = pltpu.async_copy(w_hbm.at[pl.ds(row0, SUBC)],
                                      wbuf.at[b], sem_l[b])
                d3.wait(); d2.wait(); d1.wait()
                for j in range(SUBC):
                    pltpu.async_copy(h_hbm.at[gbuf.at[b, j]],
                                     rows_v.at[b, pl.ds(j * CHUNK, CHUNK)],
                                     sem_g[b])

        def process(k, b):
            ch = wid + k * NW
            @pl.when(ch < nblocks)
            def _():
                for j in range(SUBC):
                    pltpu.make_async_copy(
                        h_hbm.at[gbuf.at[b, j]],
                        rows_v.at[b, pl.ds(j * CHUNK, CHUNK)],
                        sem_g[b]).wait()

                def scale(gi, _):
                    j = gi // (CHUNK // LANES)
                    off = (gi % (CHUNK // LANES)) * LANES
                    w16 = wbuf[b, j, pl.ds(off, LANES)]
                    for l in range(LANES):
                        wv = w16[l]
                        e = gi * LANES + l
                        for f in range(D // LANES):
                            sl = pl.ds(f * LANES, LANES)
                            rows_v[b, e, sl] = rows_v[b, e, sl] * wv
                    return 0
                lax.fori_loop(0, BLK // LANES, scale, 0)

                for j in range(SUBC):
                    pltpu.sync_copy(rows_v.at[b, pl.ds(j * CHUNK, CHUNK)],
                                    acc_sp.at[dbuf.at[b, j]], add=True)

        prefetch(jnp.int32(0), 0)

        def step(t, _):
            k0 = t * 2
            prefetch(k0 + 1, 1)
            process(k0, 0)
            prefetch(k0 + 2, 0)
            process(k0 + 1, 1)
            return 0
        lax.fori_loop(0, half, step, 0)
        plsc.subcore_barrier()

        def drain(i, _):
            k = s + i * NS
            @pl.when(k < nsub)
            def _():
                pltpu.sync_copy(acc_sp.at[pl.ds(k * sub, sub)], zbuf_v)
                pltpu.sync_copy(zbuf_v, out_hbm.at[c, pl.ds(k * sub, sub)])
            return 0
        lax.fori_loop(0, subiters, drain, 0)

    return pl.kernel(
        body,
        out_type=jax.ShapeDtypeStruct((NC, N, D), jnp.float32),
        mesh=mesh,
        scratch_types=[
            pltpu.VMEM_SHARED((N, D), jnp.float32),
            pltpu.VMEM((2, BLK, D), jnp.float32),
            pltpu.VMEM((sub, D), jnp.float32),
            pltpu.VMEM((2, SUBC, CHUNK), jnp.int32),
            pltpu.VMEM((2, SUBC, CHUNK), jnp.int32),
            pltpu.VMEM((2, SUBC, CHUNK), jnp.float32),
            pltpu.SemaphoreType.DMA,
            pltpu.SemaphoreType.DMA,
            pltpu.SemaphoreType.DMA,
            pltpu.SemaphoreType.DMA,
        ],
    )


@jax.jit
def _impl(x_author, x_paper, x_term, edge_index, edge_type,
          Wa, ba, Wp, bp, Wt, bt,
          basis1, comp1, root1, bias1,
          basis2, comp2, root2, bias2):
    num_out = x_author.shape[0]
    N = x_author.shape[0] + x_paper.shape[0] + x_term.shape[0]
    E = edge_index.shape[1]
    d_in = root1.shape[0]

    w1 = _comp_basis(comp1, basis1.reshape(basis1.shape[0], -1))
    w1all = jnp.concatenate([w1.reshape(NR, d_in, D), root1[None]], axis=0)
    w2 = _comp_basis(comp2, basis2.reshape(basis2.shape[0], -1))
    w2all = jnp.concatenate([w2.reshape(NR, d_in, D), root2[None]], axis=0)

    xa = _mm_bias(x_author, Wa, ba[None, :])
    xp = _mm_bias(x_paper, Wp, bp[None, :])
    xt = _mm_bias(x_term, Wt, bt[None, :])
    x = jnp.concatenate([xa, xp, xt], axis=0)

    src = edge_index[0]
    dst = edge_index[1]
    w_e, g_e = _edge_prep_kernel(E, N)(src, dst, edge_type)
    g2 = g_e.reshape(E // CHUNK, CHUNK)
    d2 = dst.reshape(E // CHUNK, CHUNK)
    w2 = w_e.reshape(E // CHUNK, CHUNK)

    h1 = _h_all(x, w1all, bias1[None, :])
    acc1 = _msg_kernel(E, N)(h1.reshape((NR + 1) * N, D), g2, d2, w2)
    x2 = _add3(h1[NR], acc1[0], acc1[1])

    h2 = _h_all(x2, w2all, bias2[None, :])
    acc2 = _msg_kernel(E, N)(h2.reshape((NR + 1) * N, D), g2, d2, w2)
    out = _add3(h2[NR, :num_out], acc2[0, :num_out], acc2[1, :num_out])
    return out


def kernel(x_author, x_paper, x_term, edge_index, edge_type,
           Wa, ba, Wp, bp, Wt, bt,
           basis1, comp1, root1, bias1,
           basis2, comp2, root2, bias2):
    return _impl(x_author, x_paper, x_term, edge_index, edge_type,
                 Wa, ba, Wp, bp, Wt, bt,
                 basis1, comp1, root1, bias1,
                 basis2, comp2, root2, bias2)


# speedup vs baseline: 21.7486x; 1.4656x over previous
import functools

import jax
import jax.numpy as jnp
from jax import lax
from jax.experimental import pallas as pl
from jax.experimental.pallas import tpu as pltpu
from jax.experimental.pallas import tpu_sc as plsc

NR = 5
D = 128
NC = 2
NS = 16
LANES = 16
CHUNK = 128


def _mm_bias(x, w, b2d):
    def body(x_ref, w_ref, b_ref, o_ref):
        o_ref[...] = jnp.dot(x_ref[...], w_ref[...],
                             preferred_element_type=jnp.float32) + b_ref[...]
    return pl.pallas_call(
        body,
        out_shape=jax.ShapeDtypeStruct((x.shape[0], w.shape[1]), jnp.float32),
    )(x, w, b2d)


def _comp_basis(comp, basis2d):
    def body(c_ref, b_ref, o_ref):
        o_ref[...] = jnp.dot(c_ref[...], b_ref[...],
                             preferred_element_type=jnp.float32)
    return pl.pallas_call(
        body,
        out_shape=jax.ShapeDtypeStruct((comp.shape[0], basis2d.shape[1]),
                                       jnp.float32),
    )(comp, basis2d)


def _h_all(x, wall, b2d):
    n = x.shape[0]
    blk = 1000

    def body(x_ref, w_ref, b_ref, o_ref):
        r = pl.program_id(0)
        h = jnp.dot(x_ref[...], w_ref[0], preferred_element_type=jnp.float32)
        o_ref[0] = h + jnp.where(r == NR, 1.0, 0.0) * b_ref[...]

    return pl.pallas_call(
        body,
        grid=(NR + 1, n // blk),
        in_specs=[
            pl.BlockSpec((blk, D), lambda r, i: (i, 0)),
            pl.BlockSpec((1, D, D), lambda r, i: (r, 0, 0)),
            pl.BlockSpec((1, D), lambda r, i: (0, 0)),
        ],
        out_specs=pl.BlockSpec((1, blk, D), lambda r, i: (r, i, 0)),
        out_shape=jax.ShapeDtypeStruct((NR + 1, n, D), jnp.float32),
    )(x, wall, b2d)


def _add3(a, b, c):
    n = a.shape[0]
    blk = 1000 if n % 1000 == 0 else n

    def body(a_ref, b_ref, c_ref, o_ref):
        o_ref[...] = a_ref[...] + b_ref[...] + c_ref[...]

    return pl.pallas_call(
        body,
        grid=(n // blk,),
        in_specs=[pl.BlockSpec((blk, D), lambda i: (i, 0))] * 3,
        out_specs=pl.BlockSpec((blk, D), lambda i: (i, 0)),
        out_shape=jax.ShapeDtypeStruct((n, D), jnp.float32),
    )(a, b, c)


@functools.lru_cache(maxsize=None)
def _edge_prep_kernel(E, N):
    BE = 4
    nrows = E // CHUNK
    nblk = nrows // BE
    cnt_size = NR * N
    stripe = ((cnt_size + NS * 128 - 1) // (NS * 128)) * 128
    cnt_pad = stripe * NS
    half1 = ((nblk + NS - 1) // NS + 1) // 2
    half2 = ((nblk + NS * NC - 1) // (NS * NC) + 1) // 2
    mesh = plsc.VectorSubcoreMesh(core_axis_name="c", subcore_axis_name="s")

    def body(src_hbm, dst_hbm, et_hbm, w_hbm, g_hbm,
             cnt_sp, tbl_v, ebuf, sbuf, dbuf, ixbuf, gbuf, wbuf, ones_v,
             sem_l0, sem_l1, sem_sc0, sem_sc1, sem_wg0, sem_wg1,
             sem_st0, sem_st1):
        s = lax.axis_index("s")
        c = lax.axis_index("c")
        wid = s * NC + c
        sem_l = (sem_l0, sem_l1)
        sem_sc = (sem_sc0, sem_sc1)
        sem_wg = (sem_wg0, sem_wg1)
        sem_st = (sem_st0, sem_st1)

        def z(i, _):
            tbl_v[pl.ds(i * LANES, LANES)] = jnp.zeros((LANES,), jnp.float32)
            return 0
        lax.fori_loop(0, stripe // LANES, z, 0)
        pltpu.sync_copy(tbl_v.at[pl.ds(0, stripe)],
                        cnt_sp.at[pl.ds(s * stripe, stripe)])
        plsc.subcore_barrier()

        for j in range(CHUNK // LANES):
            ones_v[pl.ds(j * LANES, LANES)] = jnp.ones((LANES,), jnp.float32)

        def pre1(k, b):
            ch = s + k * NS
            @pl.when((k >= 0) & (ch < nblk))
            def _():
                r0 = ch * BE
                pltpu.async_copy(et_hbm.at[pl.ds(r0, BE)], ebuf.at[b],
                                 sem_l[b])
                pltpu.async_copy(dst_hbm.at[pl.ds(r0, BE)], dbuf.at[b],
                                 sem_l[b])

        def drain1(k, b):
            ch = s + k * NS
            @pl.when((k >= 0) & (ch < nblk))
            def _():
                for j in range(BE):
                    pltpu.make_async_copy(ones_v,
                                          cnt_sp.at[ixbuf.at[b, j]],
                                          sem_sc[b]).wait()

        def proc1(k, b):
            ch = s + k * NS
            @pl.when((k >= 0) & (ch < nblk))
            def _():
                r0 = ch * BE
                pltpu.make_async_copy(et_hbm.at[pl.ds(r0, BE)], ebuf.at[b],
                                      sem_l[b]).wait()
                pltpu.make_async_copy(dst_hbm.at[pl.ds(r0, BE)], dbuf.at[b],
                                      sem_l[b]).wait()
                for j in range(BE):
                    for u in range(CHUNK // LANES):
                        sl = pl.ds(u * LANES, LANES)
                        ixbuf[b, j, sl] = ebuf[b, j, sl] * N + dbuf[b, j, sl]
                for j in range(BE):
                    pltpu.async_copy(ones_v, cnt_sp.at[ixbuf.at[b, j]],
                                     sem_sc[b], add=True)

        pre1(jnp.int32(0), 0)

        def step1(t, _):
            k0 = t * 2
            drain1(k0 - 2, 0)
            pre1(k0 + 1, 1)
            proc1(k0, 0)
            drain1(k0 - 1, 1)
            pre1(k0 + 2, 0)
            proc1(k0 + 1, 1)
            return 0
        lax.fori_loop(0, half1, step1, 0)
        drain1(jnp.int32(2 * half1 - 2), 0)
        drain1(jnp.int32(2 * half1 - 1), 1)
        plsc.subcore_barrier()

        pltpu.sync_copy(cnt_sp.at[pl.ds(s * stripe, stripe)],
                        tbl_v.at[pl.ds(0, stripe)])
        def inv(i, _):
            sl = pl.ds(i * LANES, LANES)
            tbl_v[sl] = 1.0 / jnp.maximum(tbl_v[sl], 1.0)
            return 0
        lax.fori_loop(0, stripe // LANES, inv, 0)
        pltpu.sync_copy(tbl_v.at[pl.ds(0, stripe)],
                        cnt_sp.at[pl.ds(s * stripe, stripe)])
        plsc.subcore_barrier()

        NW = NS * NC

        def pre2(k, b):
            ch = wid + k * NW
            @pl.when((k >= 0) & (ch < nblk))
            def _():
                r0 = ch * BE
                pltpu.async_copy(et_hbm.at[pl.ds(r0, BE)], ebuf.at[b],
                                 sem_l[b])
                pltpu.async_copy(src_hbm.at[pl.ds(r0, BE)], sbuf.at[b],
                                 sem_l[b])
                pltpu.async_copy(dst_hbm.at[pl.ds(r0, BE)], dbuf.at[b],
                                 sem_l[b])

        def drain2(k, b):
            ch = wid + k * NW
            @pl.when((k >= 0) & (ch < nblk))
            def _():
                r0 = ch * BE
                pltpu.make_async_copy(gbuf.at[b],
                                      g_hbm.at[pl.ds(r0, BE)],
                                      sem_st[b]).wait()
                pltpu.make_async_copy(wbuf.at[b],
                                      w_hbm.at[pl.ds(r0, BE)],
                                      sem_st[b]).wait()

        def proc2(k, b):
            ch = wid + k * NW
            @pl.when((k >= 0) & (ch < nblk))
            def _():
                r0 = ch * BE
                pltpu.make_async_copy(et_hbm.at[pl.ds(r0, BE)], ebuf.at[b],
                                      sem_l[b]).wait()
                pltpu.make_async_copy(src_hbm.at[pl.ds(r0, BE)], sbuf.at[b],
                                      sem_l[b]).wait()
                pltpu.make_async_copy(dst_hbm.at[pl.ds(r0, BE)], dbuf.at[b],
                                      sem_l[b]).wait()
                for j in range(BE):
                    for u in range(CHUNK // LANES):
                        sl = pl.ds(u * LANES, LANES)
                        gbuf[b, j, sl] = ebuf[b, j, sl] * N + sbuf[b, j, sl]
                        ixbuf[b, j, sl] = ebuf[b, j, sl] * N + dbuf[b, j, sl]
                pltpu.async_copy(gbuf.at[b], g_hbm.at[pl.ds(r0, BE)],
                                 sem_st[b])
                for j in range(BE):
                    pltpu.async_copy(cnt_sp.at[ixbuf.at[b, j]], wbuf.at[b, j],
                                     sem_wg[b])
                for j in range(BE):
                    pltpu.make_async_copy(cnt_sp.at[ixbuf.at[b, j]],
                                          wbuf.at[b, j], sem_wg[b]).wait()
                pltpu.async_copy(wbuf.at[b], w_hbm.at[pl.ds(r0, BE)],
                                 sem_st[b])

        pre2(jnp.int32(0), 0)

        def step2(t, _):
            k0 = t * 2
            drain2(k0 - 2, 0)
            pre2(k0 + 1, 1)
            proc2(k0, 0)
            drain2(k0 - 1, 1)
            pre2(k0 + 2, 0)
            proc2(k0 + 1, 1)
            return 0
        lax.fori_loop(0, half2, step2, 0)
        drain2(jnp.int32(2 * half2 - 2), 0)
        drain2(jnp.int32(2 * half2 - 1), 1)

    return pl.kernel(
        body,
        out_type=[jax.ShapeDtypeStruct((nrows, CHUNK), jnp.float32),
                  jax.ShapeDtypeStruct((nrows, CHUNK), jnp.int32)],
        mesh=mesh,
        scratch_types=[
            pltpu.VMEM_SHARED((cnt_pad,), jnp.float32),
            pltpu.VMEM((stripe,), jnp.float32),
            pltpu.VMEM((2, BE, CHUNK), jnp.int32),
            pltpu.VMEM((2, BE, CHUNK), jnp.int32),
            pltpu.VMEM((2, BE, CHUNK), jnp.int32),
            pltpu.VMEM((2, BE, CHUNK), jnp.int32),
            pltpu.VMEM((2, BE, CHUNK), jnp.int32),
            pltpu.VMEM((2, BE, CHUNK), jnp.float32),
            pltpu.VMEM((CHUNK,), jnp.float32),
            pltpu.SemaphoreType.DMA, pltpu.SemaphoreType.DMA,
            pltpu.SemaphoreType.DMA, pltpu.SemaphoreType.DMA,
            pltpu.SemaphoreType.DMA, pltpu.SemaphoreType.DMA,
            pltpu.SemaphoreType.DMA, pltpu.SemaphoreType.DMA,
        ],
    )


@functools.lru_cache(maxsize=None)
def _msg_kernel(E, N):
    BLK = 128
    SUBC = BLK // CHUNK
    nblocks = E // BLK
    NW = NS * NC
    iters = (nblocks + NW - 1) // NW
    half = (iters + 1) // 2
    sub = 40
    nsub = N // sub
    subiters = (nsub + NS - 1) // NS
    mesh = plsc.VectorSubcoreMesh(core_axis_name="c", subcore_axis_name="s")

    def body(h_hbm, g_hbm, dst_hbm, w_hbm, out_hbm,
             acc_sp, rows_v, zbuf_v, gbuf, dbuf, wbuf,
             sem_l0, sem_l1, sem_g0, sem_g1):
        s = lax.axis_index("s")
        c = lax.axis_index("c")
        wid = s * NC + c
        sem_l = (sem_l0, sem_l1)
        sem_g = (sem_g0, sem_g1)

        def z(i, _):
            for j in range(D // LANES):
                zbuf_v[i, pl.ds(j * LANES, LANES)] = jnp.zeros((LANES,),
                                                               jnp.float32)
            return 0
        lax.fori_loop(0, sub, z, 0)

        def z2(i, _):
            k = s + i * NS
            @pl.when(k < nsub)
            def _():
                pltpu.sync_copy(zbuf_v, acc_sp.at[pl.ds(k * sub, sub)])
            return 0
        lax.fori_loop(0, subiters, z2, 0)
        plsc.subcore_barrier()

        def prefetch(k, b):
            ch = wid + k * NW
            @pl.when(ch < nblocks)
            def _():
                row0 = ch * SUBC
                d1 = pltpu.async_copy(g_hbm.at[pl.ds(row0, SUBC)],
                                      gbuf.at[b], sem_l[b])
                d2 = pltpu.async_copy(dst_hbm.at[pl.ds(row0, SUBC)],
                                      dbuf.at[b], sem_l[b])
                d3 = pltpu.async_copy(w_hbm.at[pl.ds(row0, SUBC)],
                                      wbuf.at[b], sem_l[b])
                d3.wait(); d2.wait(); d1.wait()
                for j in range(SUBC):
                    pltpu.async_copy(h_hbm.at[gbuf.at[b, j]],
                                     rows_v.at[b, pl.ds(j * CHUNK, CHUNK)],
                                     sem_g[b])

        def process(k, b):
            ch = wid + k * NW
            @pl.when(ch < nblocks)
            def _():
                for j in range(SUBC):
                    pltpu.make_async_copy(
                        h_hbm.at[gbuf.at[b, j]],
                        rows_v.at[b, pl.ds(j * CHUNK, CHUNK)],
                        sem_g[b]).wait()

                def scale(gi, _):
                    j = gi // (CHUNK // LANES)
                    off = (gi % (CHUNK // LANES)) * LANES
                    w16 = wbuf[b, j, pl.ds(off, LANES)]
                    for l in range(LANES):
                        wv = w16[l]
                        e = gi * LANES + l
                        for f in range(D // LANES):
                            sl = pl.ds(f * LANES, LANES)
                            rows_v[b, e, sl] = rows_v[b, e, sl] * wv
                    return 0
                lax.fori_loop(0, BLK // LANES, scale, 0)

                for j in range(SUBC):
                    pltpu.sync_copy(rows_v.at[b, pl.ds(j * CHUNK, CHUNK)],
                                    acc_sp.at[dbuf.at[b, j]], add=True)

        prefetch(jnp.int32(0), 0)

        def step(t, _):
            k0 = t * 2
            prefetch(k0 + 1, 1)
            process(k0, 0)
            prefetch(k0 + 2, 0)
            process(k0 + 1, 1)
            return 0
        lax.fori_loop(0, half, step, 0)
        plsc.subcore_barrier()

        def drain(i, _):
            k = s + i * NS
            @pl.when(k < nsub)
            def _():
                pltpu.sync_copy(acc_sp.at[pl.ds(k * sub, sub)], zbuf_v)
                pltpu.sync_copy(zbuf_v, out_hbm.at[c, pl.ds(k * sub, sub)])
            return 0
        lax.fori_loop(0, subiters, drain, 0)

    return pl.kernel(
        body,
        out_type=jax.ShapeDtypeStruct((NC, N, D), jnp.float32),
        mesh=mesh,
        scratch_types=[
            pltpu.VMEM_SHARED((N, D), jnp.float32),
            pltpu.VMEM((2, BLK, D), jnp.float32),
            pltpu.VMEM((sub, D), jnp.float32),
            pltpu.VMEM((2, SUBC, CHUNK), jnp.int32),
            pltpu.VMEM((2, SUBC, CHUNK), jnp.int32),
            pltpu.VMEM((2, SUBC, CHUNK), jnp.float32),
            pltpu.SemaphoreType.DMA,
            pltpu.SemaphoreType.DMA,
            pltpu.SemaphoreType.DMA,
            pltpu.SemaphoreType.DMA,
        ],
    )


@jax.jit
def _impl(x_author, x_paper, x_term, edge_index, edge_type,
          Wa, ba, Wp, bp, Wt, bt,
          basis1, comp1, root1, bias1,
          basis2, comp2, root2, bias2):
    num_out = x_author.shape[0]
    N = x_author.shape[0] + x_paper.shape[0] + x_term.shape[0]
    E = edge_index.shape[1]
    d_in = root1.shape[0]

    w1 = _comp_basis(comp1, basis1.reshape(basis1.shape[0], -1))
    w1all = jnp.concatenate([w1.reshape(NR, d_in, D), root1[None]], axis=0)
    w2 = _comp_basis(comp2, basis2.reshape(basis2.shape[0], -1))
    w2all = jnp.concatenate([w2.reshape(NR, d_in, D), root2[None]], axis=0)

    xa = _mm_bias(x_author, Wa, ba[None, :])
    xp = _mm_bias(x_paper, Wp, bp[None, :])
    xt = _mm_bias(x_term, Wt, bt[None, :])
    x = jnp.concatenate([xa, xp, xt], axis=0)

    src = edge_index[0].reshape(E // CHUNK, CHUNK)
    d2 = edge_index[1].reshape(E // CHUNK, CHUNK)
    et2 = edge_type.reshape(E // CHUNK, CHUNK)
    w2, g2 = _edge_prep_kernel(E, N)(src, d2, et2)

    h1 = _h_all(x, w1all, bias1[None, :])
    acc1 = _msg_kernel(E, N)(h1.reshape((NR + 1) * N, D), g2, d2, w2)
    x2 = _add3(h1[NR], acc1[0], acc1[1])

    h2 = _h_all(x2, w2all, bias2[None, :])
    acc2 = _msg_kernel(E, N)(h2.reshape((NR + 1) * N, D), g2, d2, w2)
    out = _add3(h2[NR, :num_out], acc2[0, :num_out], acc2[1, :num_out])
    return out


def kernel(x_author, x_paper, x_term, edge_index, edge_type,
           Wa, ba, Wp, bp, Wt, bt,
           basis1, comp1, root1, bias1,
           basis2, comp2, root2, bias2):
    return _impl(x_author, x_paper, x_term, edge_index, edge_type,
                 Wa, ba, Wp, bp, Wt, bt,
                 basis1, comp1, root1, bias1,
                 basis2, comp2, root2, bias2)


# speedup vs baseline: 23.6141x; 1.0858x over previous
import functools

import jax
import jax.numpy as jnp
from jax import lax
from jax.experimental import pallas as pl
from jax.experimental.pallas import tpu as pltpu
from jax.experimental.pallas import tpu_sc as plsc

NR = 5
D = 128
NC = 2
NS = 16
LANES = 16
CHUNK = 128


def _mm_bias(x, w, b2d):
    def body(x_ref, w_ref, b_ref, o_ref):
        o_ref[...] = jnp.dot(x_ref[...], w_ref[...],
                             preferred_element_type=jnp.float32) + b_ref[...]
    return pl.pallas_call(
        body,
        out_shape=jax.ShapeDtypeStruct((x.shape[0], w.shape[1]), jnp.float32),
    )(x, w, b2d)


def _comp_basis(comp, basis2d):
    def body(c_ref, b_ref, o_ref):
        o_ref[...] = jnp.dot(c_ref[...], b_ref[...],
                             preferred_element_type=jnp.float32)
    return pl.pallas_call(
        body,
        out_shape=jax.ShapeDtypeStruct((comp.shape[0], basis2d.shape[1]),
                                       jnp.float32),
    )(comp, basis2d)


def _h_all(x, wall, b2d):
    n = x.shape[0]
    blk = 1000

    def body(x_ref, w_ref, b_ref, o_ref):
        r = pl.program_id(0)
        h = jnp.dot(x_ref[...], w_ref[0], preferred_element_type=jnp.float32)
        o_ref[0] = h + jnp.where(r == NR, 1.0, 0.0) * b_ref[...]

    return pl.pallas_call(
        body,
        grid=(NR + 1, n // blk),
        in_specs=[
            pl.BlockSpec((blk, D), lambda r, i: (i, 0)),
            pl.BlockSpec((1, D, D), lambda r, i: (r, 0, 0)),
            pl.BlockSpec((1, D), lambda r, i: (0, 0)),
        ],
        out_specs=pl.BlockSpec((1, blk, D), lambda r, i: (r, i, 0)),
        out_shape=jax.ShapeDtypeStruct((NR + 1, n, D), jnp.float32),
    )(x, wall, b2d)


def _h_all_sum(base, a0, a1, wall, b2d):
    n = base.shape[0]
    blk = 1000

    def body(x_ref, a0_ref, a1_ref, w_ref, b_ref, o_ref):
        x2 = x_ref[...] + a0_ref[...] + a1_ref[...]
        for r in range(NR + 1):
            h = jnp.dot(x2, w_ref[r], preferred_element_type=jnp.float32)
            o_ref[r] = h + b_ref[...] if r == NR else h

    return pl.pallas_call(
        body,
        grid=(n // blk,),
        in_specs=[
            pl.BlockSpec((blk, D), lambda i: (i, 0)),
            pl.BlockSpec((blk, D), lambda i: (i, 0)),
            pl.BlockSpec((blk, D), lambda i: (i, 0)),
            pl.BlockSpec((NR + 1, D, D), lambda i: (0, 0, 0)),
            pl.BlockSpec((1, D), lambda i: (0, 0)),
        ],
        out_specs=pl.BlockSpec((NR + 1, blk, D), lambda i: (0, i, 0)),
        out_shape=jax.ShapeDtypeStruct((NR + 1, n, D), jnp.float32),
    )(base, a0, a1, wall, b2d)


def _add3(a, b, c):
    n = a.shape[0]
    blk = 1000 if n % 1000 == 0 else n

    def body(a_ref, b_ref, c_ref, o_ref):
        o_ref[...] = a_ref[...] + b_ref[...] + c_ref[...]

    return pl.pallas_call(
        body,
        grid=(n // blk,),
        in_specs=[pl.BlockSpec((blk, D), lambda i: (i, 0))] * 3,
        out_specs=pl.BlockSpec((blk, D), lambda i: (i, 0)),
        out_shape=jax.ShapeDtypeStruct((n, D), jnp.float32),
    )(a, b, c)


@functools.lru_cache(maxsize=None)
def _edge_prep_kernel(E, N):
    BE = 4
    nrows = E // CHUNK
    nblk = nrows // BE
    cnt_size = NR * N
    stripe = ((cnt_size + NS * 128 - 1) // (NS * 128)) * 128
    cnt_pad = stripe * NS
    half1 = ((nblk + NS - 1) // NS + 1) // 2
    half2 = ((nblk + NS * NC - 1) // (NS * NC) + 1) // 2
    mesh = plsc.VectorSubcoreMesh(core_axis_name="c", subcore_axis_name="s")

    def body(src_hbm, dst_hbm, et_hbm, w_hbm, g_hbm,
             cnt_sp, tbl_v, ebuf, sbuf, dbuf, ixbuf, gbuf, wbuf, ones_v,
             sem_l0, sem_l1, sem_sc0, sem_sc1, sem_wg0, sem_wg1,
             sem_st0, sem_st1):
        s = lax.axis_index("s")
        c = lax.axis_index("c")
        wid = s * NC + c
        sem_l = (sem_l0, sem_l1)
        sem_sc = (sem_sc0, sem_sc1)
        sem_wg = (sem_wg0, sem_wg1)
        sem_st = (sem_st0, sem_st1)

        def z(i, _):
            tbl_v[pl.ds(i * LANES, LANES)] = jnp.zeros((LANES,), jnp.float32)
            return 0
        lax.fori_loop(0, stripe // LANES, z, 0)
        pltpu.sync_copy(tbl_v.at[pl.ds(0, stripe)],
                        cnt_sp.at[pl.ds(s * stripe, stripe)])
        plsc.subcore_barrier()

        for j in range(CHUNK // LANES):
            ones_v[pl.ds(j * LANES, LANES)] = jnp.ones((LANES,), jnp.float32)

        def pre1(k, b):
            ch = s + k * NS
            @pl.when((k >= 0) & (ch < nblk))
            def _():
                r0 = ch * BE
                pltpu.async_copy(et_hbm.at[pl.ds(r0, BE)], ebuf.at[b],
                                 sem_l[b])
                pltpu.async_copy(dst_hbm.at[pl.ds(r0, BE)], dbuf.at[b],
                                 sem_l[b])

        def drain1(k, b):
            ch = s + k * NS
            @pl.when((k >= 0) & (ch < nblk))
            def _():
                for j in range(BE):
                    pltpu.make_async_copy(ones_v,
                                          cnt_sp.at[ixbuf.at[b, j]],
                                          sem_sc[b]).wait()

        def proc1(k, b):
            ch = s + k * NS
            @pl.when((k >= 0) & (ch < nblk))
            def _():
                r0 = ch * BE
                pltpu.make_async_copy(et_hbm.at[pl.ds(r0, BE)], ebuf.at[b],
                                      sem_l[b]).wait()
                pltpu.make_async_copy(dst_hbm.at[pl.ds(r0, BE)], dbuf.at[b],
                                      sem_l[b]).wait()
                for j in range(BE):
                    for u in range(CHUNK // LANES):
                        sl = pl.ds(u * LANES, LANES)
                        ixbuf[b, j, sl] = ebuf[b, j, sl] * N + dbuf[b, j, sl]
                for j in range(BE):
                    pltpu.async_copy(ones_v, cnt_sp.at[ixbuf.at[b, j]],
                                     sem_sc[b], add=True)

        pre1(jnp.int32(0), 0)

        def step1(t, _):
            k0 = t * 2
            drain1(k0 - 2, 0)
            pre1(k0 + 1, 1)
            proc1(k0, 0)
            drain1(k0 - 1, 1)
            pre1(k0 + 2, 0)
            proc1(k0 + 1, 1)
            return 0
        lax.fori_loop(0, half1, step1, 0)
        drain1(jnp.int32(2 * half1 - 2), 0)
        drain1(jnp.int32(2 * half1 - 1), 1)
        plsc.subcore_barrier()

        pltpu.sync_copy(cnt_sp.at[pl.ds(s * stripe, stripe)],
                        tbl_v.at[pl.ds(0, stripe)])
        def inv(i, _):
            sl = pl.ds(i * LANES, LANES)
            tbl_v[sl] = 1.0 / jnp.maximum(tbl_v[sl], 1.0)
            return 0
        lax.fori_loop(0, stripe // LANES, inv, 0)
        pltpu.sync_copy(tbl_v.at[pl.ds(0, stripe)],
                        cnt_sp.at[pl.ds(s * stripe, stripe)])
        plsc.subcore_barrier()

        NW = NS * NC

        def pre2(k, b):
            ch = wid + k * NW
            @pl.when((k >= 0) & (ch < nblk))
            def _():
                r0 = ch * BE
                pltpu.async_copy(et_hbm.at[pl.ds(r0, BE)], ebuf.at[b],
                                 sem_l[b])
                pltpu.async_copy(src_hbm.at[pl.ds(r0, BE)], sbuf.at[b],
                                 sem_l[b])
                pltpu.async_copy(dst_hbm.at[pl.ds(r0, BE)], dbuf.at[b],
                                 sem_l[b])

        def drain2(k, b):
            ch = wid + k * NW
            @pl.when((k >= 0) & (ch < nblk))
            def _():
                r0 = ch * BE
                pltpu.make_async_copy(gbuf.at[b],
                                      g_hbm.at[pl.ds(r0, BE)],
                                      sem_st[b]).wait()
                pltpu.make_async_copy(wbuf.at[b],
                                      w_hbm.at[pl.ds(r0, BE)],
                                      sem_st[b]).wait()

        def proc2(k, b):
            ch = wid + k * NW
            @pl.when((k >= 0) & (ch < nblk))
            def _():
                r0 = ch * BE
                pltpu.make_async_copy(et_hbm.at[pl.ds(r0, BE)], ebuf.at[b],
                                      sem_l[b]).wait()
                pltpu.make_async_copy(src_hbm.at[pl.ds(r0, BE)], sbuf.at[b],
                                      sem_l[b]).wait()
                pltpu.make_async_copy(dst_hbm.at[pl.ds(r0, BE)], dbuf.at[b],
                                      sem_l[b]).wait()
                for j in range(BE):
                    for u in range(CHUNK // LANES):
                        sl = pl.ds(u * LANES, LANES)
                        gbuf[b, j, sl] = ebuf[b, j, sl] * N + sbuf[b, j, sl]
                        ixbuf[b, j, sl] = ebuf[b, j, sl] * N + dbuf[b, j, sl]
                pltpu.async_copy(gbuf.at[b], g_hbm.at[pl.ds(r0, BE)],
                                 sem_st[b])
                for j in range(BE):
                    pltpu.async_copy(cnt_sp.at[ixbuf.at[b, j]], wbuf.at[b, j],
                                     sem_wg[b])
                for j in range(BE):
                    pltpu.make_async_copy(cnt_sp.at[ixbuf.at[b, j]],
                                          wbuf.at[b, j], sem_wg[b]).wait()
                pltpu.async_copy(wbuf.at[b], w_hbm.at[pl.ds(r0, BE)],
                                 sem_st[b])

        pre2(jnp.int32(0), 0)

        def step2(t, _):
            k0 = t * 2
            drain2(k0 - 2, 0)
            pre2(k0 + 1, 1)
            proc2(k0, 0)
            drain2(k0 - 1, 1)
            pre2(k0 + 2, 0)
            proc2(k0 + 1, 1)
            return 0
        lax.fori_loop(0, half2, step2, 0)
        drain2(jnp.int32(2 * half2 - 2), 0)
        drain2(jnp.int32(2 * half2 - 1), 1)

    return pl.kernel(
        body,
        out_type=[jax.ShapeDtypeStruct((nrows, CHUNK), jnp.float32),
                  jax.ShapeDtypeStruct((nrows, CHUNK), jnp.int32)],
        mesh=mesh,
        scratch_types=[
            pltpu.VMEM_SHARED((cnt_pad,), jnp.float32),
            pltpu.VMEM((stripe,), jnp.float32),
            pltpu.VMEM((2, BE, CHUNK), jnp.int32),
            pltpu.VMEM((2, BE, CHUNK), jnp.int32),
            pltpu.VMEM((2, BE, CHUNK), jnp.int32),
            pltpu.VMEM((2, BE, CHUNK), jnp.int32),
            pltpu.VMEM((2, BE, CHUNK), jnp.int32),
            pltpu.VMEM((2, BE, CHUNK), jnp.float32),
            pltpu.VMEM((CHUNK,), jnp.float32),
            pltpu.SemaphoreType.DMA, pltpu.SemaphoreType.DMA,
            pltpu.SemaphoreType.DMA, pltpu.SemaphoreType.DMA,
            pltpu.SemaphoreType.DMA, pltpu.SemaphoreType.DMA,
            pltpu.SemaphoreType.DMA, pltpu.SemaphoreType.DMA,
        ],
    )


@functools.lru_cache(maxsize=None)
def _msg_kernel(E, N):
    BLK = 128
    SUBC = BLK // CHUNK
    nblocks = E // BLK
    NW = NS * NC
    iters = (nblocks + NW - 1) // NW
    half = (iters + 1) // 2
    sub = 40
    nsub = N // sub
    subiters = (nsub + NS - 1) // NS
    mesh = plsc.VectorSubcoreMesh(core_axis_name="c", subcore_axis_name="s")

    def body(h_hbm, g_hbm, dst_hbm, w_hbm, out_hbm,
             acc_sp, rows_v, zbuf_v, gbuf, dbuf, wbuf,
             sem_l0, sem_l1, sem_g0, sem_g1):
        s = lax.axis_index("s")
        c = lax.axis_index("c")
        wid = s * NC + c
        sem_l = (sem_l0, sem_l1)
        sem_g = (sem_g0, sem_g1)

        def z(i, _):
            for j in range(D // LANES):
                zbuf_v[i, pl.ds(j * LANES, LANES)] = jnp.zeros((LANES,),
                                                               jnp.float32)
            return 0
        lax.fori_loop(0, sub, z, 0)

        def z2(i, _):
            k = s + i * NS
            @pl.when(k < nsub)
            def _():
                pltpu.sync_copy(zbuf_v, acc_sp.at[pl.ds(k * sub, sub)])
            return 0
        lax.fori_loop(0, subiters, z2, 0)
        plsc.subcore_barrier()

        def prefetch(k, b):
            ch = wid + k * NW
            @pl.when(ch < nblocks)
            def _():
                row0 = ch * SUBC
                d1 = pltpu.async_copy(g_hbm.at[pl.ds(row0, SUBC)],
                                      gbuf.at[b], sem_l[b])
                d2 = pltpu.async_copy(dst_hbm.at[pl.ds(row0, SUBC)],
                                      dbuf.at[b], sem_l[b])
                d3 = pltpu.async_copy(w_hbm.at[pl.ds(row0, SUBC)],
                                      wbuf.at[b], sem_l[b])
                d3.wait(); d2.wait(); d1.wait()
                for j in range(SUBC):
                    pltpu.async_copy(h_hbm.at[gbuf.at[b, j]],
                                     rows_v.at[b, pl.ds(j * CHUNK, CHUNK)],
                                     sem_g[b])

        def process(k, b):
            ch = wid + k * NW
            @pl.when(ch < nblocks)
            def _():
                for j in range(SUBC):
                    pltpu.make_async_copy(
                        h_hbm.at[gbuf.at[b, j]],
                        rows_v.at[b, pl.ds(j * CHUNK, CHUNK)],
                        sem_g[b]).wait()

                def scale(gi, _):
                    j = gi // (CHUNK // LANES)
                    off = (gi % (CHUNK // LANES)) * LANES
                    w16 = wbuf[b, j, pl.ds(off, LANES)]
                    for l in range(LANES):
                        wv = w16[l]
                        e = gi * LANES + l
                        for f in range(D // LANES):
                            sl = pl.ds(f * LANES, LANES)
                            rows_v[b, e, sl] = rows_v[b, e, sl] * wv
                    return 0
                lax.fori_loop(0, BLK // LANES, scale, 0)

                for j in range(SUBC):
                    pltpu.sync_copy(rows_v.at[b, pl.ds(j * CHUNK, CHUNK)],
                                    acc_sp.at[dbuf.at[b, j]], add=True)

        prefetch(jnp.int32(0), 0)

        def step(t, _):
            k0 = t * 2
            prefetch(k0 + 1, 1)
            process(k0, 0)
            prefetch(k0 + 2, 0)
            process(k0 + 1, 1)
            return 0
        lax.fori_loop(0, half, step, 0)
        plsc.subcore_barrier()

        def drain(i, _):
            k = s + i * NS
            @pl.when(k < nsub)
            def _():
                pltpu.sync_copy(acc_sp.at[pl.ds(k * sub, sub)], zbuf_v)
                pltpu.sync_copy(zbuf_v, out_hbm.at[c, pl.ds(k * sub, sub)])
            return 0
        lax.fori_loop(0, subiters, drain, 0)

    return pl.kernel(
        body,
        out_type=jax.ShapeDtypeStruct((NC, N, D), jnp.float32),
        mesh=mesh,
        scratch_types=[
            pltpu.VMEM_SHARED((N, D), jnp.float32),
            pltpu.VMEM((2, BLK, D), jnp.float32),
            pltpu.VMEM((sub, D), jnp.float32),
            pltpu.VMEM((2, SUBC, CHUNK), jnp.int32),
            pltpu.VMEM((2, SUBC, CHUNK), jnp.int32),
            pltpu.VMEM((2, SUBC, CHUNK), jnp.float32),
            pltpu.SemaphoreType.DMA,
            pltpu.SemaphoreType.DMA,
            pltpu.SemaphoreType.DMA,
            pltpu.SemaphoreType.DMA,
        ],
    )


@jax.jit
def _impl(x_author, x_paper, x_term, edge_index, edge_type,
          Wa, ba, Wp, bp, Wt, bt,
          basis1, comp1, root1, bias1,
          basis2, comp2, root2, bias2):
    num_out = x_author.shape[0]
    N = x_author.shape[0] + x_paper.shape[0] + x_term.shape[0]
    E = edge_index.shape[1]
    d_in = root1.shape[0]

    w1 = _comp_basis(comp1, basis1.reshape(basis1.shape[0], -1))
    w1all = jnp.concatenate([w1.reshape(NR, d_in, D), root1[None]], axis=0)
    w2 = _comp_basis(comp2, basis2.reshape(basis2.shape[0], -1))
    w2all = jnp.concatenate([w2.reshape(NR, d_in, D), root2[None]], axis=0)

    xa = _mm_bias(x_author, Wa, ba[None, :])
    xp = _mm_bias(x_paper, Wp, bp[None, :])
    xt = _mm_bias(x_term, Wt, bt[None, :])
    x = jnp.concatenate([xa, xp, xt], axis=0)

    src = edge_index[0].reshape(E // CHUNK, CHUNK)
    d2 = edge_index[1].reshape(E // CHUNK, CHUNK)
    et2 = edge_type.reshape(E // CHUNK, CHUNK)
    w2, g2 = _edge_prep_kernel(E, N)(src, d2, et2)

    h1 = _h_all(x, w1all, bias1[None, :])
    acc1 = _msg_kernel(E, N)(h1.reshape((NR + 1) * N, D), g2, d2, w2)

    h2 = _h_all_sum(h1[NR], acc1[0], acc1[1], w2all, bias2[None, :])
    acc2 = _msg_kernel(E, N)(h2.reshape((NR + 1) * N, D), g2, d2, w2)
    out = _add3(h2[NR, :num_out], acc2[0, :num_out], acc2[1, :num_out])
    return out


def kernel(x_author, x_paper, x_term, edge_index, edge_type,
           Wa, ba, Wp, bp, Wt, bt,
           basis1, comp1, root1, bias1,
           basis2, comp2, root2, bias2):
    return _impl(x_author, x_paper, x_term, edge_index, edge_type,
                 Wa, ba, Wp, bp, Wt, bt,
                 basis1, comp1, root1, bias1,
                 basis2, comp2, root2, bias2)


# speedup vs baseline: 25.3641x; 1.0741x over previous
import functools

import jax
import jax.numpy as jnp
from jax import lax
from jax.experimental import pallas as pl
from jax.experimental.pallas import tpu as pltpu
from jax.experimental.pallas import tpu_sc as plsc

NR = 5
D = 128
NC = 2
NS = 16
LANES = 16
CHUNK = 128


def _mm_bias(x, w, b2d):
    def body(x_ref, w_ref, b_ref, o_ref):
        o_ref[...] = jnp.dot(x_ref[...], w_ref[...],
                             preferred_element_type=jnp.float32) + b_ref[...]
    return pl.pallas_call(
        body,
        out_shape=jax.ShapeDtypeStruct((x.shape[0], w.shape[1]), jnp.float32),
    )(x, w, b2d)


def _comp_basis(comp, basis2d):
    def body(c_ref, b_ref, o_ref):
        o_ref[...] = jnp.dot(c_ref[...], b_ref[...],
                             preferred_element_type=jnp.float32)
    return pl.pallas_call(
        body,
        out_shape=jax.ShapeDtypeStruct((comp.shape[0], basis2d.shape[1]),
                                       jnp.float32),
    )(comp, basis2d)


def _h_all(x, wall, b2d):
    n = x.shape[0]
    blk = 1000

    def body(x_ref, w_ref, b_ref, o_ref):
        x1 = x_ref[...]
        for r in range(NR + 1):
            h = jnp.dot(x1, w_ref[r], preferred_element_type=jnp.float32)
            o_ref[r] = h + b_ref[...] if r == NR else h

    return pl.pallas_call(
        body,
        grid=(n // blk,),
        in_specs=[
            pl.BlockSpec((blk, D), lambda i: (i, 0)),
            pl.BlockSpec((NR + 1, D, D), lambda i: (0, 0, 0)),
            pl.BlockSpec((1, D), lambda i: (0, 0)),
        ],
        out_specs=pl.BlockSpec((NR + 1, blk, D), lambda i: (0, i, 0)),
        out_shape=jax.ShapeDtypeStruct((NR + 1, n, D), jnp.float32),
    )(x, wall, b2d)


def _h_all_sum(base, a0, a1, wall, b2d):
    n = base.shape[0]
    blk = 1000

    def body(x_ref, a0_ref, a1_ref, w_ref, b_ref, o_ref):
        x2 = x_ref[...] + a0_ref[...] + a1_ref[...]
        for r in range(NR + 1):
            h = jnp.dot(x2, w_ref[r], preferred_element_type=jnp.float32)
            o_ref[r] = h + b_ref[...] if r == NR else h

    return pl.pallas_call(
        body,
        grid=(n // blk,),
        in_specs=[
            pl.BlockSpec((blk, D), lambda i: (i, 0)),
            pl.BlockSpec((blk, D), lambda i: (i, 0)),
            pl.BlockSpec((blk, D), lambda i: (i, 0)),
            pl.BlockSpec((NR + 1, D, D), lambda i: (0, 0, 0)),
            pl.BlockSpec((1, D), lambda i: (0, 0)),
        ],
        out_specs=pl.BlockSpec((NR + 1, blk, D), lambda i: (0, i, 0)),
        out_shape=jax.ShapeDtypeStruct((NR + 1, n, D), jnp.float32),
    )(base, a0, a1, wall, b2d)


def _add3(a, b, c):
    n = a.shape[0]
    blk = 1000 if n % 1000 == 0 else n

    def body(a_ref, b_ref, c_ref, o_ref):
        o_ref[...] = a_ref[...] + b_ref[...] + c_ref[...]

    return pl.pallas_call(
        body,
        grid=(n // blk,),
        in_specs=[pl.BlockSpec((blk, D), lambda i: (i, 0))] * 3,
        out_specs=pl.BlockSpec((blk, D), lambda i: (i, 0)),
        out_shape=jax.ShapeDtypeStruct((n, D), jnp.float32),
    )(a, b, c)


@functools.lru_cache(maxsize=None)
def _edge_prep_kernel(E, N):
    BE = 4
    nrows = E // CHUNK
    nblk = nrows // BE
    cnt_size = NR * N
    stripe = ((cnt_size + NS * 128 - 1) // (NS * 128)) * 128
    cnt_pad = stripe * NS
    half1 = ((nblk + NS - 1) // NS + 1) // 2
    half2 = ((nblk + NS * NC - 1) // (NS * NC) + 1) // 2
    mesh = plsc.VectorSubcoreMesh(core_axis_name="c", subcore_axis_name="s")

    def body(src_hbm, dst_hbm, et_hbm, w_hbm, g_hbm,
             cnt_sp, tbl_v, ebuf, sbuf, dbuf, ixbuf, gbuf, wbuf, ones_v,
             sem_l0, sem_l1, sem_sc0, sem_sc1, sem_wg0, sem_wg1,
             sem_st0, sem_st1):
        s = lax.axis_index("s")
        c = lax.axis_index("c")
        wid = s * NC + c
        sem_l = (sem_l0, sem_l1)
        sem_sc = (sem_sc0, sem_sc1)
        sem_wg = (sem_wg0, sem_wg1)
        sem_st = (sem_st0, sem_st1)

        def z(i, _):
            tbl_v[pl.ds(i * LANES, LANES)] = jnp.zeros((LANES,), jnp.float32)
            return 0
        lax.fori_loop(0, stripe // LANES, z, 0)
        pltpu.sync_copy(tbl_v.at[pl.ds(0, stripe)],
                        cnt_sp.at[pl.ds(s * stripe, stripe)])
        plsc.subcore_barrier()

        for j in range(CHUNK // LANES):
            ones_v[pl.ds(j * LANES, LANES)] = jnp.ones((LANES,), jnp.float32)

        def pre1(k, b):
            ch = s + k * NS
            @pl.when((k >= 0) & (ch < nblk))
            def _():
                r0 = ch * BE
                pltpu.async_copy(et_hbm.at[pl.ds(r0, BE)], ebuf.at[b],
                                 sem_l[b])
                pltpu.async_copy(dst_hbm.at[pl.ds(r0, BE)], dbuf.at[b],
                                 sem_l[b])

        def drain1(k, b):
            ch = s + k * NS
            @pl.when((k >= 0) & (ch < nblk))
            def _():
                for j in range(BE):
                    pltpu.make_async_copy(ones_v,
                                          cnt_sp.at[ixbuf.at[b, j]],
                                          sem_sc[b]).wait()

        def proc1(k, b):
            ch = s + k * NS
            @pl.when((k >= 0) & (ch < nblk))
            def _():
                r0 = ch * BE
                pltpu.make_async_copy(et_hbm.at[pl.ds(r0, BE)], ebuf.at[b],
                                      sem_l[b]).wait()
                pltpu.make_async_copy(dst_hbm.at[pl.ds(r0, BE)], dbuf.at[b],
                                      sem_l[b]).wait()
                for j in range(BE):
                    for u in range(CHUNK // LANES):
                        sl = pl.ds(u * LANES, LANES)
                        ixbuf[b, j, sl] = ebuf[b, j, sl] * N + dbuf[b, j, sl]
                for j in range(BE):
                    pltpu.async_copy(ones_v, cnt_sp.at[ixbuf.at[b, j]],
                                     sem_sc[b], add=True)

        pre1(jnp.int32(0), 0)

        def step1(t, _):
            k0 = t * 2
            drain1(k0 - 2, 0)
            pre1(k0 + 1, 1)
            proc1(k0, 0)
            drain1(k0 - 1, 1)
            pre1(k0 + 2, 0)
            proc1(k0 + 1, 1)
            return 0
        lax.fori_loop(0, half1, step1, 0)
        drain1(jnp.int32(2 * half1 - 2), 0)
        drain1(jnp.int32(2 * half1 - 1), 1)
        plsc.subcore_barrier()

        pltpu.sync_copy(cnt_sp.at[pl.ds(s * stripe, stripe)],
                        tbl_v.at[pl.ds(0, stripe)])
        def inv(i, _):
            sl = pl.ds(i * LANES, LANES)
            tbl_v[sl] = 1.0 / jnp.maximum(tbl_v[sl], 1.0)
            return 0
        lax.fori_loop(0, stripe // LANES, inv, 0)
        pltpu.sync_copy(tbl_v.at[pl.ds(0, stripe)],
                        cnt_sp.at[pl.ds(s * stripe, stripe)])
        plsc.subcore_barrier()

        NW = NS * NC

        def pre2(k, b):
            ch = wid + k * NW
            @pl.when((k >= 0) & (ch < nblk))
            def _():
                r0 = ch * BE
                pltpu.async_copy(et_hbm.at[pl.ds(r0, BE)], ebuf.at[b],
                                 sem_l[b])
                pltpu.async_copy(src_hbm.at[pl.ds(r0, BE)], sbuf.at[b],
                                 sem_l[b])
                pltpu.async_copy(dst_hbm.at[pl.ds(r0, BE)], dbuf.at[b],
                                 sem_l[b])

        def drain2(k, b):
            ch = wid + k * NW
            @pl.when((k >= 0) & (ch < nblk))
            def _():
                r0 = ch * BE
                pltpu.make_async_copy(gbuf.at[b],
                                      g_hbm.at[pl.ds(r0, BE)],
                                      sem_st[b]).wait()
                pltpu.make_async_copy(wbuf.at[b],
                                      w_hbm.at[pl.ds(r0, BE)],
                                      sem_st[b]).wait()

        def proc2(k, b):
            ch = wid + k * NW
            @pl.when((k >= 0) & (ch < nblk))
            def _():
                r0 = ch * BE
                pltpu.make_async_copy(et_hbm.at[pl.ds(r0, BE)], ebuf.at[b],
                                      sem_l[b]).wait()
                pltpu.make_async_copy(src_hbm.at[pl.ds(r0, BE)], sbuf.at[b],
                                      sem_l[b]).wait()
                pltpu.make_async_copy(dst_hbm.at[pl.ds(r0, BE)], dbuf.at[b],
                                      sem_l[b]).wait()
                for j in range(BE):
                    for u in range(CHUNK // LANES):
                        sl = pl.ds(u * LANES, LANES)
                        gbuf[b, j, sl] = ebuf[b, j, sl] * N + sbuf[b, j, sl]
                        ixbuf[b, j, sl] = ebuf[b, j, sl] * N + dbuf[b, j, sl]
                pltpu.async_copy(gbuf.at[b], g_hbm.at[pl.ds(r0, BE)],
                                 sem_st[b])
                for j in range(BE):
                    pltpu.async_copy(cnt_sp.at[ixbuf.at[b, j]], wbuf.at[b, j],
                                     sem_wg[b])
                for j in range(BE):
                    pltpu.make_async_copy(cnt_sp.at[ixbuf.at[b, j]],
                                          wbuf.at[b, j], sem_wg[b]).wait()
                pltpu.async_copy(wbuf.at[b], w_hbm.at[pl.ds(r0, BE)],
                                 sem_st[b])

        pre2(jnp.int32(0), 0)

        def step2(t, _):
            k0 = t * 2
            drain2(k0 - 2, 0)
            pre2(k0 + 1, 1)
            proc2(k0, 0)
            drain2(k0 - 1, 1)
            pre2(k0 + 2, 0)
            proc2(k0 + 1, 1)
            return 0
        lax.fori_loop(0, half2, step2, 0)
        drain2(jnp.int32(2 * half2 - 2), 0)
        drain2(jnp.int32(2 * half2 - 1), 1)

    return pl.kernel(
        body,
        out_type=[jax.ShapeDtypeStruct((nrows, CHUNK), jnp.float32),
                  jax.ShapeDtypeStruct((nrows, CHUNK), jnp.int32)],
        mesh=mesh,
        scratch_types=[
            pltpu.VMEM_SHARED((cnt_pad,), jnp.float32),
            pltpu.VMEM((stripe,), jnp.float32),
            pltpu.VMEM((2, BE, CHUNK), jnp.int32),
            pltpu.VMEM((2, BE, CHUNK), jnp.int32),
            pltpu.VMEM((2, BE, CHUNK), jnp.int32),
            pltpu.VMEM((2, BE, CHUNK), jnp.int32),
            pltpu.VMEM((2, BE, CHUNK), jnp.int32),
            pltpu.VMEM((2, BE, CHUNK), jnp.float32),
            pltpu.VMEM((CHUNK,), jnp.float32),
            pltpu.SemaphoreType.DMA, pltpu.SemaphoreType.DMA,
            pltpu.SemaphoreType.DMA, pltpu.SemaphoreType.DMA,
            pltpu.SemaphoreType.DMA, pltpu.SemaphoreType.DMA,
            pltpu.SemaphoreType.DMA, pltpu.SemaphoreType.DMA,
        ],
    )


@functools.lru_cache(maxsize=None)
def _msg_kernel(E, N):
    BLK = 128
    SUBC = BLK // CHUNK
    nblocks = E // BLK
    NW = NS * NC
    iters = (nblocks + NW - 1) // NW
    half = (iters + 1) // 2
    sub = 40
    nsub = N // sub
    subiters = (nsub + NS - 1) // NS
    mesh = plsc.VectorSubcoreMesh(core_axis_name="c", subcore_axis_name="s")

    def body(h_hbm, g_hbm, dst_hbm, w_hbm, out_hbm,
             acc_sp, rows_v, zbuf_v, gbuf, dbuf, wbuf,
             sem_l0, sem_l1, sem_g0, sem_g1):
        s = lax.axis_index("s")
        c = lax.axis_index("c")
        wid = s * NC + c
        sem_l = (sem_l0, sem_l1)
        sem_g = (sem_g0, sem_g1)

        def z(i, _):
            for j in range(D // LANES):
                zbuf_v[i, pl.ds(j * LANES, LANES)] = jnp.zeros((LANES,),
                                                               jnp.float32)
            return 0
        lax.fori_loop(0, sub, z, 0)

        def z2(i, _):
            k = s + i * NS
            @pl.when(k < nsub)
            def _():
                pltpu.sync_copy(zbuf_v, acc_sp.at[pl.ds(k * sub, sub)])
            return 0
        lax.fori_loop(0, subiters, z2, 0)
        plsc.subcore_barrier()

        def prefetch(k, b):
            ch = wid + k * NW
            @pl.when(ch < nblocks)
            def _():
                row0 = ch * SUBC
                d1 = pltpu.async_copy(g_hbm.at[pl.ds(row0, SUBC)],
                                      gbuf.at[b], sem_l[b])
                d2 = pltpu.async_copy(dst_hbm.at[pl.ds(row0, SUBC)],
                                      dbuf.at[b], sem_l[b])
                d3 = pltpu.async_copy(w_hbm.at[pl.ds(row0, SUBC)],
                                      wbuf.at[b], sem_l[b])
                d3.wait(); d2.wait(); d1.wait()
                for j in range(SUBC):
                    pltpu.async_copy(h_hbm.at[gbuf.at[b, j]],
                                     rows_v.at[b, pl.ds(j * CHUNK, CHUNK)],
                                     sem_g[b])

        def process(k, b):
            ch = wid + k * NW
            @pl.when(ch < nblocks)
            def _():
                for j in range(SUBC):
                    pltpu.make_async_copy(
                        h_hbm.at[gbuf.at[b, j]],
                        rows_v.at[b, pl.ds(j * CHUNK, CHUNK)],
                        sem_g[b]).wait()

                def scale(gi, _):
                    j = gi // (CHUNK // LANES)
                    off = (gi % (CHUNK // LANES)) * LANES
                    w16 = wbuf[b, j, pl.ds(off, LANES)]
                    for l in range(LANES):
                        wv = w16[l]
                        e = gi * LANES + l
                        for f in range(D // LANES):
                            sl = pl.ds(f * LANES, LANES)
                            rows_v[b, e, sl] = rows_v[b, e, sl] * wv
                    return 0
                lax.fori_loop(0, BLK // LANES, scale, 0)

                for j in range(SUBC):
                    pltpu.sync_copy(rows_v.at[b, pl.ds(j * CHUNK, CHUNK)],
                                    acc_sp.at[dbuf.at[b, j]], add=True)

        prefetch(jnp.int32(0), 0)

        def step(t, _):
            k0 = t * 2
            prefetch(k0 + 1, 1)
            process(k0, 0)
            prefetch(k0 + 2, 0)
            process(k0 + 1, 1)
            return 0
        lax.fori_loop(0, half, step, 0)
        plsc.subcore_barrier()

        def drain(i, _):
            k = s + i * NS
            @pl.when(k < nsub)
            def _():
                pltpu.sync_copy(acc_sp.at[pl.ds(k * sub, sub)], zbuf_v)
                pltpu.sync_copy(zbuf_v, out_hbm.at[c, pl.ds(k * sub, sub)])
            return 0
        lax.fori_loop(0, subiters, drain, 0)

    return pl.kernel(
        body,
        out_type=jax.ShapeDtypeStruct((NC, N, D), jnp.float32),
        mesh=mesh,
        scratch_types=[
            pltpu.VMEM_SHARED((N, D), jnp.float32),
            pltpu.VMEM((2, BLK, D), jnp.float32),
            pltpu.VMEM((sub, D), jnp.float32),
            pltpu.VMEM((2, SUBC, CHUNK), jnp.int32),
            pltpu.VMEM((2, SUBC, CHUNK), jnp.int32),
            pltpu.VMEM((2, SUBC, CHUNK), jnp.float32),
            pltpu.SemaphoreType.DMA,
            pltpu.SemaphoreType.DMA,
            pltpu.SemaphoreType.DMA,
            pltpu.SemaphoreType.DMA,
        ],
    )


@jax.jit
def _impl(x_author, x_paper, x_term, edge_index, edge_type,
          Wa, ba, Wp, bp, Wt, bt,
          basis1, comp1, root1, bias1,
          basis2, comp2, root2, bias2):
    num_out = x_author.shape[0]
    N = x_author.shape[0] + x_paper.shape[0] + x_term.shape[0]
    E = edge_index.shape[1]
    d_in = root1.shape[0]

    w1 = _comp_basis(comp1, basis1.reshape(basis1.shape[0], -1))
    w1all = jnp.concatenate([w1.reshape(NR, d_in, D), root1[None]], axis=0)
    w2 = _comp_basis(comp2, basis2.reshape(basis2.shape[0], -1))
    w2all = jnp.concatenate([w2.reshape(NR, d_in, D), root2[None]], axis=0)

    xa = _mm_bias(x_author, Wa, ba[None, :])
    xp = _mm_bias(x_paper, Wp, bp[None, :])
    xt = _mm_bias(x_term, Wt, bt[None, :])
    x = jnp.concatenate([xa, xp, xt], axis=0)

    src = edge_index[0].reshape(E // CHUNK, CHUNK)
    d2 = edge_index[1].reshape(E // CHUNK, CHUNK)
    et2 = edge_type.reshape(E // CHUNK, CHUNK)
    w2, g2 = _edge_prep_kernel(E, N)(src, d2, et2)

    h1 = _h_all(x, w1all, bias1[None, :])
    acc1 = _msg_kernel(E, N)(h1.reshape((NR + 1) * N, D), g2, d2, w2)

    h2 = _h_all_sum(h1[NR], acc1[0], acc1[1], w2all, bias2[None, :])
    acc2 = _msg_kernel(E, N)(h2.reshape((NR + 1) * N, D), g2, d2, w2)
    out = _add3(h2[NR, :num_out], acc2[0, :num_out], acc2[1, :num_out])
    return out


def kernel(x_author, x_paper, x_term, edge_index, edge_type,
           Wa, ba, Wp, bp, Wt, bt,
           basis1, comp1, root1, bias1,
           basis2, comp2, root2, bias2):
    return _impl(x_author, x_paper, x_term, edge_index, edge_type,
                 Wa, ba, Wp, bp, Wt, bt,
                 basis1, comp1, root1, bias1,
                 basis2, comp2, root2, bias2)
